# baseline (speedup 1.0000x reference)
import jax
import jax.numpy as jnp
from jax.experimental import pallas as pl
from jax.experimental.pallas import tpu as pltpu

_N = 125000
_S = 62500
_K = 1000
_ROWS = 1024
_NEG = float("-inf")
_BIG = 2**30


def _kernel(c0_ref, c1_ref, c2_ref, packed_ref,
            scores_ref, boxes_ref, dir_ref,
            m_s, rowmax_s, anch_s, delta_s):
    m = jnp.maximum(jnp.maximum(c0_ref[:], c1_ref[:]), c2_ref[:])
    m_s[:] = m
    rowmax_s[:] = jnp.max(m.reshape(8, 128, 128), axis=2)

    lane_iota = jax.lax.broadcasted_iota(jnp.int32, (1, 128), 1)
    flat_iota = (jax.lax.broadcasted_iota(jnp.int32, (8, 128), 0) * 128
                 + jax.lax.broadcasted_iota(jnp.int32, (8, 128), 1))

    def body(i, _):
        rm = rowmax_s[:]
        g = jnp.max(rm)
        r = jnp.min(jnp.where(rm == g, flat_iota, _BIG))
        row = m_s[pl.ds(r, 1), :]
        lane = jnp.min(jnp.where(row == g, lane_iota, _BIG))
        n = r * 128 + lane

        newrow = jnp.where(lane_iota == lane, _NEG, row)
        m_s[pl.ds(r, 1), :] = newrow
        ri = jax.lax.div(r, 128)
        rj = jax.lax.rem(r, 128)
        rmrow = rowmax_s[pl.ds(ri, 1), :]
        rowmax_s[pl.ds(ri, 1), :] = jnp.where(lane_iota == rj,
                                              jnp.max(newrow), rmrow)

        g4 = jax.lax.div(n, 4)
        sub = jax.lax.rem(n, 4)
        prow = packed_ref[pl.ds(g4, 1), :]
        grp = jnp.where(sub == 1, prow[:, 19:38], prow[:, 0:19])
        grp = jnp.where(sub == 2, prow[:, 38:57], grp)
        grp = jnp.where(sub == 3, prow[:, 57:76], grp)
        scores_ref[pl.ds(i, 1), :] = jax.nn.sigmoid(grp[:, 0:3])
        delta_s[pl.ds(i, 1), :] = grp[:, 3:10]
        dir_ref[pl.ds(i, 1), :] = (grp[:, 11:12] > grp[:, 10:11]
                                   ).astype(jnp.int32)
        anch_s[pl.ds(i, 1), :] = grp[:, 12:19]
        return 0

    jax.lax.fori_loop(0, _K, body, 0)

    an = anch_s[:]
    dl = delta_s[:]
    xa, ya, za = an[:, 0:1], an[:, 1:2], an[:, 2:3]
    wa, la, ha, ra = an[:, 3:4], an[:, 4:5], an[:, 5:6], an[:, 6:7]
    xt, yt, zt = dl[:, 0:1], dl[:, 1:2], dl[:, 2:3]
    wt, lt, ht, rt = dl[:, 3:4], dl[:, 4:5], dl[:, 5:6], dl[:, 6:7]
    za = za + ha * 0.5
    diag = jnp.sqrt(la * la + wa * wa)
    xg = xt * diag + xa
    yg = yt * diag + ya
    hg = jnp.exp(ht) * ha
    zg = zt * ha + za - hg * 0.5
    lg = jnp.exp(lt) * la
    wg = jnp.exp(wt) * wa
    rg = rt + ra
    boxes_ref[:] = jnp.concatenate([xg, yg, zg, wg, lg, hg, rg], axis=1)


@jax.jit
def kernel(cls_score, bbox_pred, dir_cls_pred, anchors_fixed):
    cls_n = jnp.transpose(cls_score.reshape(6, _S)).reshape(_N, 3)
    bbox_n = jnp.transpose(bbox_pred.reshape(14, _S)).reshape(_N, 7)
    dir_n = jnp.transpose(dir_cls_pred.reshape(4, _S)).reshape(_N, 2)
    packed = jnp.concatenate(
        [cls_n, bbox_n, dir_n, anchors_fixed], axis=1).reshape(31250, 76)

    pad = _ROWS * 128 - _N
    planes = [
        jnp.pad(cls_n[:, c], (0, pad),
                constant_values=-jnp.inf).reshape(_ROWS, 128)
        for c in range(3)
    ]

    scores, boxes, dircls = pl.pallas_call(
        _kernel,
        out_shape=[
            jax.ShapeDtypeStruct((_K, 3), jnp.float32),
            jax.ShapeDtypeStruct((_K, 7), jnp.float32),
            jax.ShapeDtypeStruct((_K, 1), jnp.int32),
        ],
        scratch_shapes=[
            pltpu.VMEM((_ROWS, 128), jnp.float32),
            pltpu.VMEM((8, 128), jnp.float32),
            pltpu.VMEM((_K, 7), jnp.float32),
            pltpu.VMEM((_K, 7), jnp.float32),
        ],
    )(planes[0], planes[1], planes[2], packed)

    return (scores, boxes, dircls.reshape(_K))


# speedup vs baseline: 1.0004x; 1.0004x over previous
import jax
import jax.numpy as jnp
from jax.experimental import pallas as pl
from jax.experimental.pallas import tpu as pltpu

_N = 125000
_S = 62500
_K = 1000
_ROWS = 1024
_NEG = float("-inf")
_BIG = 2**30


def _kernel(c0_ref, c1_ref, c2_ref, packed_ref,
            scores_ref, boxes_ref, dir_ref,
            m_s, anch_s, delta_s):
    m = jnp.maximum(jnp.maximum(c0_ref[:], c1_ref[:]), c2_ref[:])
    m_s[:] = m

    lane_iota = jax.lax.broadcasted_iota(jnp.int32, (1, 128), 1)
    flat_iota = (jax.lax.broadcasted_iota(jnp.int32, (8, 128), 0) * 128
                 + jax.lax.broadcasted_iota(jnp.int32, (8, 128), 1))

    def body(i, rm):
        g = jnp.max(rm)
        r = jnp.min(jnp.where(rm == g, flat_iota, _BIG))
        row = m_s[pl.ds(r, 1), :]
        lane = jnp.min(jnp.where(row == g, lane_iota, _BIG))

        newrow = jnp.where(lane_iota == lane, _NEG, row)
        m_s[pl.ds(r, 1), :] = newrow
        rm = jnp.where(flat_iota == r, jnp.max(newrow), rm)

        g4 = r * 32 + jax.lax.div(lane, 4)
        sub = jax.lax.rem(lane, 4)
        prow = packed_ref[pl.ds(g4, 1), :]
        grp = jnp.where(sub == 1, prow[:, 19:38], prow[:, 0:19])
        grp = jnp.where(sub == 2, prow[:, 38:57], grp)
        grp = jnp.where(sub == 3, prow[:, 57:76], grp)
        scores_ref[pl.ds(i, 1), :] = jax.nn.sigmoid(grp[:, 0:3])
        delta_s[pl.ds(i, 1), :] = grp[:, 3:10]
        dir_ref[pl.ds(i, 1), :] = (grp[:, 11:12] > grp[:, 10:11]
                                   ).astype(jnp.int32)
        anch_s[pl.ds(i, 1), :] = grp[:, 12:19]
        return rm

    rm0 = jnp.max(m.reshape(8, 128, 128), axis=2)
    jax.lax.fori_loop(0, _K, body, rm0)

    an = anch_s[:]
    dl = delta_s[:]
    xa, ya, za = an[:, 0:1], an[:, 1:2], an[:, 2:3]
    wa, la, ha, ra = an[:, 3:4], an[:, 4:5], an[:, 5:6], an[:, 6:7]
    xt, yt, zt = dl[:, 0:1], dl[:, 1:2], dl[:, 2:3]
    wt, lt, ht, rt = dl[:, 3:4], dl[:, 4:5], dl[:, 5:6], dl[:, 6:7]
    za = za + ha * 0.5
    diag = jnp.sqrt(la * la + wa * wa)
    xg = xt * diag + xa
    yg = yt * diag + ya
    hg = jnp.exp(ht) * ha
    zg = zt * ha + za - hg * 0.5
    lg = jnp.exp(lt) * la
    wg = jnp.exp(wt) * wa
    rg = rt + ra
    boxes_ref[:] = jnp.concatenate([xg, yg, zg, wg, lg, hg, rg], axis=1)


@jax.jit
def kernel(cls_score, bbox_pred, dir_cls_pred, anchors_fixed):
    cls_n = jnp.transpose(cls_score.reshape(6, _S)).reshape(_N, 3)
    bbox_n = jnp.transpose(bbox_pred.reshape(14, _S)).reshape(_N, 7)
    dir_n = jnp.transpose(dir_cls_pred.reshape(4, _S)).reshape(_N, 2)
    packed = jnp.concatenate(
        [cls_n, bbox_n, dir_n, anchors_fixed], axis=1).reshape(31250, 76)

    pad = _ROWS * 128 - _N
    planes = [
        jnp.pad(cls_n[:, c], (0, pad),
                constant_values=-jnp.inf).reshape(_ROWS, 128)
        for c in range(3)
    ]

    scores, boxes, dircls = pl.pallas_call(
        _kernel,
        out_shape=[
            jax.ShapeDtypeStruct((_K, 3), jnp.float32),
            jax.ShapeDtypeStruct((_K, 7), jnp.float32),
            jax.ShapeDtypeStruct((_K, 1), jnp.int32),
        ],
        scratch_shapes=[
            pltpu.VMEM((_ROWS, 128), jnp.float32),
            pltpu.VMEM((_K, 7), jnp.float32),
            pltpu.VMEM((_K, 7), jnp.float32),
        ],
    )(planes[0], planes[1], planes[2], packed)

    return (scores, boxes, dircls.reshape(_K))


# speedup vs baseline: 1.1449x; 1.1444x over previous
import jax
import jax.numpy as jnp
from jax.experimental import pallas as pl
from jax.experimental.pallas import tpu as pltpu

_N = 125000
_S = 62500
_K = 1000
_ROWS = 1024
_NEG = float("-inf")
_BIG = 2**30


def _kernel(c0_ref, c1_ref, c2_ref, packed_ref,
            scores_ref, boxes_ref, dir_ref,
            m_s, anch_s, delta_s):
    m = jnp.maximum(jnp.maximum(c0_ref[:], c1_ref[:]), c2_ref[:])
    m_s[:] = m

    lane_iota = jax.lax.broadcasted_iota(jnp.int32, (1, 128), 1)
    flat_iota = (jax.lax.broadcasted_iota(jnp.int32, (8, 128), 0) * 128
                 + jax.lax.broadcasted_iota(jnp.int32, (8, 128), 1))

    def step(i, rm):
        g = jnp.max(rm)
        r = jnp.min(jnp.where(rm == g, flat_iota, _BIG))
        row = m_s[pl.ds(r, 1), :]
        lane = jnp.min(jnp.where(row == g, lane_iota, _BIG))

        newrow = jnp.where(lane_iota == lane, _NEG, row)
        m_s[pl.ds(r, 1), :] = newrow
        rm = jnp.where(flat_iota == r, jnp.max(newrow), rm)

        g4 = r * 32 + jax.lax.div(lane, 4)
        sub = jax.lax.rem(lane, 4)
        prow = packed_ref[pl.ds(g4, 1), :]
        grp = jnp.where(sub == 1, prow[:, 19:38], prow[:, 0:19])
        grp = jnp.where(sub == 2, prow[:, 38:57], grp)
        grp = jnp.where(sub == 3, prow[:, 57:76], grp)
        scores_ref[pl.ds(i, 1), :] = jax.nn.sigmoid(grp[:, 0:3])
        delta_s[pl.ds(i, 1), :] = grp[:, 3:10]
        dir_ref[pl.ds(i, 1), :] = (grp[:, 11:12] > grp[:, 10:11]
                                   ).astype(jnp.int32)
        anch_s[pl.ds(i, 1), :] = grp[:, 12:19]
        return rm

    def body(j, rm):
        i = j * 4
        for u in range(4):
            rm = step(i + u, rm)
        return rm

    rm0 = jnp.max(m.reshape(8, 128, 128), axis=2)
    jax.lax.fori_loop(0, _K // 4, body, rm0)

    an = anch_s[:]
    dl = delta_s[:]
    xa, ya, za = an[:, 0:1], an[:, 1:2], an[:, 2:3]
    wa, la, ha, ra = an[:, 3:4], an[:, 4:5], an[:, 5:6], an[:, 6:7]
    xt, yt, zt = dl[:, 0:1], dl[:, 1:2], dl[:, 2:3]
    wt, lt, ht, rt = dl[:, 3:4], dl[:, 4:5], dl[:, 5:6], dl[:, 6:7]
    za = za + ha * 0.5
    diag = jnp.sqrt(la * la + wa * wa)
    xg = xt * diag + xa
    yg = yt * diag + ya
    hg = jnp.exp(ht) * ha
    zg = zt * ha + za - hg * 0.5
    lg = jnp.exp(lt) * la
    wg = jnp.exp(wt) * wa
    rg = rt + ra
    boxes_ref[:] = jnp.concatenate([xg, yg, zg, wg, lg, hg, rg], axis=1)


@jax.jit
def kernel(cls_score, bbox_pred, dir_cls_pred, anchors_fixed):
    cls_n = jnp.transpose(cls_score.reshape(6, _S)).reshape(_N, 3)
    bbox_n = jnp.transpose(bbox_pred.reshape(14, _S)).reshape(_N, 7)
    dir_n = jnp.transpose(dir_cls_pred.reshape(4, _S)).reshape(_N, 2)
    packed = jnp.concatenate(
        [cls_n, bbox_n, dir_n, anchors_fixed], axis=1).reshape(31250, 76)

    pad = _ROWS * 128 - _N
    planes = [
        jnp.pad(cls_n[:, c], (0, pad),
                constant_values=-jnp.inf).reshape(_ROWS, 128)
        for c in range(3)
    ]

    scores, boxes, dircls = pl.pallas_call(
        _kernel,
        out_shape=[
            jax.ShapeDtypeStruct((_K, 3), jnp.float32),
            jax.ShapeDtypeStruct((_K, 7), jnp.float32),
            jax.ShapeDtypeStruct((_K, 1), jnp.int32),
        ],
        scratch_shapes=[
            pltpu.VMEM((_ROWS, 128), jnp.float32),
            pltpu.VMEM((_K, 7), jnp.float32),
            pltpu.VMEM((_K, 7), jnp.float32),
        ],
    )(planes[0], planes[1], planes[2], packed)

    return (scores, boxes, dircls.reshape(_K))
